# Initial kernel scaffold; baseline (speedup 1.0000x reference)
import jax
import jax.numpy as jnp
from jax.experimental import pallas as pl


def kernel(x, edge_index, batch, W0, b0, gamma0, beta0, W1, b1, gamma1, beta1, W2, b2, gamma2, beta2):
    raise NotImplementedError("write your pallas kernel here")


# speedup vs baseline: 18.6078x; 18.6078x over previous
import functools

import jax
import jax.numpy as jnp
from jax import lax
from jax.experimental import pallas as pl
from jax.experimental.pallas import tpu as pltpu
from jax.experimental.pallas import tpu_sc as plsc

N = 10000
E = 320000
D = 128
G = 64
EPS = 1e-5

NC = 2
NS = 16
EPC = E // NC
EPT = E // (NC * NS)
C = 80
NCHUNK = EPT // C
NPAD = 10240
RPT = NPAD // NS

_mesh = plsc.VectorSubcoreMesh(core_axis_name="c", subcore_axis_name="s")


def _copy_idx(src_ref, src_off, dst_ref, n):
    for i in range(n // 16):
        dst_ref[pl.ds(i * 16, 16)] = src_ref[pl.ds(src_off + i * 16, 16)]


DEGW = D


@functools.partial(
    pl.kernel,
    mesh=_mesh,
    out_type=jax.ShapeDtypeStruct((NC * NPAD, DEGW), jnp.float32),
    scratch_types=[
        pltpu.VMEM((C,), jnp.int32),
        pltpu.VMEM((C, DEGW), jnp.float32),
        pltpu.VMEM_SHARED((NPAD, DEGW), jnp.float32),
    ],
)
def _sc_deg(dst_hbm, ones_hbm, zeros_hbm, out_hbm, dst_v, ones_v, acc_sh):
    cid = lax.axis_index("c")
    sid = lax.axis_index("s")
    pltpu.sync_copy(zeros_hbm.at[pl.ds(sid * RPT, RPT)],
                    acc_sh.at[pl.ds(sid * RPT, RPT)])
    pltpu.sync_copy(ones_hbm, ones_v)
    plsc.subcore_barrier()
    base = cid * EPC + sid * EPT

    def body(j, carry):
        off = base + j * C
        pltpu.sync_copy(dst_hbm.at[pl.ds(off, C)], dst_v)
        pltpu.sync_copy(ones_v, acc_sh.at[dst_v], add=True)
        return carry

    lax.fori_loop(0, NCHUNK, body, 0)
    plsc.subcore_barrier()
    pltpu.sync_copy(acc_sh.at[pl.ds(sid * RPT, RPT)],
                    out_hbm.at[pl.ds(cid * NPAD + sid * RPT, RPT)])


CF = 96
NFULL = EPT // CF
TAIL = EPT - NFULL * CF


@functools.partial(
    pl.kernel,
    mesh=_mesh,
    out_type=jax.ShapeDtypeStruct((NC * NPAD, D), jnp.float32),
    scratch_types=[
        pltpu.VMEM((EPT,), jnp.int32),
        pltpu.VMEM((EPT,), jnp.int32),
        pltpu.VMEM((CF,), jnp.int32),
        pltpu.VMEM((CF,), jnp.int32),
        pltpu.VMEM((TAIL,), jnp.int32),
        pltpu.VMEM((CF, D), jnp.float32),
        pltpu.VMEM((CF, D), jnp.float32),
        pltpu.VMEM_SHARED((NPAD, D), jnp.float32),
        pltpu.SemaphoreType.DMA,
        pltpu.SemaphoreType.DMA,
    ],
)
def _sc_mp(g_hbm, src_hbm, dst_hbm, zeros_hbm, out_hbm,
           src_all, dst_all, gidx, sidx, idx_t, rows0, rows1, acc_sh,
           sem0, sem1):
    cid = lax.axis_index("c")
    sid = lax.axis_index("s")
    base = cid * EPC + sid * EPT
    pltpu.sync_copy(src_hbm.at[pl.ds(base, EPT)], src_all)
    pltpu.sync_copy(dst_hbm.at[pl.ds(base, EPT)], dst_all)
    pltpu.sync_copy(zeros_hbm.at[pl.ds(sid * RPT, RPT)],
                    acc_sh.at[pl.ds(sid * RPT, RPT)])
    plsc.subcore_barrier()

    rows = (rows0, rows1)
    sems = (sem0, sem1)

    def issue_gather(j, p):
        _copy_idx(src_all, j * CF, gidx, CF)
        return pltpu.async_copy(g_hbm.at[gidx], rows[p], sems[p])

    def scatter(j, p):
        _copy_idx(dst_all, j * CF, sidx, CF)
        pltpu.sync_copy(rows[p], acc_sh.at[sidx], add=True)

    issue_gather(0, 0).wait()

    def pair(k, carry):
        j = 2 * k + 1
        h = issue_gather(j, 1)
        scatter(j - 1, 0)
        h.wait()
        h = issue_gather(j + 1, 0)
        scatter(j, 1)
        h.wait()
        return carry

    lax.fori_loop(0, (NFULL - 2) // 2, pair, 0)
    h = issue_gather(NFULL - 1, 1)
    scatter(NFULL - 2, 0)
    h.wait()
    scatter(NFULL - 1, 1)
    pltpu.async_copy(g_hbm.at[src_all.at[pl.ds(NFULL * CF, TAIL)]],
                     rows0.at[pl.ds(0, TAIL)], sems[0]).wait()
    _copy_idx(dst_all, NFULL * CF, idx_t, TAIL)
    pltpu.sync_copy(rows0.at[pl.ds(0, TAIL)], acc_sh.at[idx_t], add=True)

    plsc.subcore_barrier()
    pltpu.sync_copy(acc_sh.at[pl.ds(sid * RPT, RPT)],
                    out_hbm.at[pl.ds(cid * NPAD + sid * RPT, RPT)])


def _mm0_body(x_ref, w_ref, degp_ref, g_ref):
    deg = 1.0 + degp_ref[0, :N, :1] + degp_ref[1, :N, :1]
    dinv = 1.0 / jnp.sqrt(deg)
    g_ref[...] = dinv * jnp.dot(x_ref[...], w_ref[...],
                                preferred_element_type=jnp.float32)


_mm0 = pl.pallas_call(
    _mm0_body,
    out_shape=jax.ShapeDtypeStruct((N, D), jnp.float32),
)


def _epi_body(with_next, s_ref, g_ref, degp_ref, b_ref, gamma_ref, beta_ref,
              batch_ref, w_ref, pool_ref, gnext_ref=None):
    deg = 1.0 + degp_ref[0, :N, :1] + degp_ref[1, :N, :1]
    dinv = 1.0 / jnp.sqrt(deg)
    s = s_ref[0, :N] + s_ref[1, :N] + g_ref[...]
    u = jnp.maximum(dinv * s + b_ref[...], 0.0)
    mean = jnp.mean(u, axis=0, keepdims=True)
    var = jnp.mean((u - mean) ** 2, axis=0, keepdims=True)
    h = gamma_ref[...] * (u - mean) / jnp.sqrt(var + EPS) + beta_ref[...]
    sel = (batch_ref[...] ==
           lax.broadcasted_iota(jnp.int32, (G, N), 0)).astype(jnp.float32)
    pool_ref[...] = jnp.dot(sel, h, preferred_element_type=jnp.float32,
                            precision=lax.Precision.HIGHEST)
    if with_next:
        gnext_ref[...] = dinv * jnp.dot(h, w_ref[...],
                                        preferred_element_type=jnp.float32)


_epi_mid = pl.pallas_call(
    functools.partial(_epi_body, True),
    out_shape=(jax.ShapeDtypeStruct((G, D), jnp.float32),
               jax.ShapeDtypeStruct((N, D), jnp.float32)),
)

_epi_last = pl.pallas_call(
    functools.partial(_epi_body, False),
    out_shape=jax.ShapeDtypeStruct((G, D), jnp.float32),
)


def kernel(x, edge_index, batch,
           W0, b0, gamma0, beta0,
           W1, b1, gamma1, beta1,
           W2, b2, gamma2, beta2):
    src = edge_index[0]
    dst = edge_index[1]
    ones_c = jnp.ones((C, DEGW), jnp.float32)
    zeros_deg = jnp.zeros((NPAD, DEGW), jnp.float32)
    zeros_nd = jnp.zeros((NPAD, D), jnp.float32)
    batch2 = batch.reshape(1, N)

    degp = _sc_deg(dst, ones_c, zeros_deg).reshape(NC, NPAD, DEGW)

    g = _mm0(x, W0, degp)

    params = [(b0, gamma0, beta0, W1), (b1, gamma1, beta1, W2),
              (b2, gamma2, beta2, None)]
    pools = []
    for li, (b, gamma, beta, Wn) in enumerate(params):
        s = _sc_mp(g, src, dst, zeros_nd).reshape(NC, NPAD, D)
        b2d = b.reshape(1, D)
        g2d = gamma.reshape(1, D)
        bt2d = beta.reshape(1, D)
        if Wn is not None:
            pool, g = _epi_mid(s, g, degp, b2d, g2d, bt2d, batch2, Wn)
        else:
            wd = jnp.zeros((D, D), jnp.float32)
            pool = _epi_last(s, g, degp, b2d, g2d, bt2d, batch2, wd)
        pools.append(pool)

    return jnp.concatenate(pools, axis=1)


# speedup vs baseline: 20.3939x; 1.0960x over previous
import functools

import jax
import jax.numpy as jnp
from jax import lax
from jax.experimental import pallas as pl
from jax.experimental.pallas import tpu as pltpu
from jax.experimental.pallas import tpu_sc as plsc

N = 10000
E = 320000
D = 128
G = 64
EPS = 1e-5

NC = 2
NS = 16
EPC = E // NC
EPT = E // (NC * NS)
C = 80
NCHUNK = EPT // C
NPAD = 10240
RPT = NPAD // NS

_mesh = plsc.VectorSubcoreMesh(core_axis_name="c", subcore_axis_name="s")


def _copy_idx(src_ref, src_off, dst_ref, n):
    for i in range(n // 16):
        dst_ref[pl.ds(i * 16, 16)] = src_ref[pl.ds(src_off + i * 16, 16)]


DEGW = D
DC = 96
DNF = EPT // DC
DTAIL = EPT - DNF * DC
DK = 4


@functools.partial(
    pl.kernel,
    mesh=_mesh,
    out_type=jax.ShapeDtypeStruct((NC * NPAD, DEGW), jnp.float32),
    scratch_types=[
        pltpu.VMEM((EPT,), jnp.int32),
        pltpu.VMEM((DC,), jnp.int32),
        pltpu.VMEM((DC,), jnp.int32),
        pltpu.VMEM((DC,), jnp.int32),
        pltpu.VMEM((DC,), jnp.int32),
        pltpu.VMEM((DTAIL,), jnp.int32),
        pltpu.VMEM((DC, DEGW), jnp.float32),
        pltpu.VMEM_SHARED((NPAD, DEGW), jnp.float32),
        pltpu.SemaphoreType.DMA,
    ],
)
def _sc_deg(dst_hbm, ones_hbm, zeros_hbm, out_hbm,
            dst_all, i0, i1, i2, i3, it, ones_v, acc_sh, sem):
    cid = lax.axis_index("c")
    sid = lax.axis_index("s")
    base = cid * EPC + sid * EPT
    pltpu.sync_copy(dst_hbm.at[pl.ds(base, EPT)], dst_all)
    pltpu.sync_copy(zeros_hbm.at[pl.ds(sid * RPT, RPT)],
                    acc_sh.at[pl.ds(sid * RPT, RPT)])
    pltpu.sync_copy(ones_hbm, ones_v)
    plsc.subcore_barrier()
    ibufs = (i0, i1, i2, i3)

    def group(kk, carry):
        j0 = kk * DK
        hs = []
        for p in range(DK):
            _copy_idx(dst_all, (j0 + p) * DC, ibufs[p], DC)
            hs.append(pltpu.async_copy(ones_v, acc_sh.at[ibufs[p]], sem,
                                       add=True))
        for h in hs:
            h.wait()
        return carry

    lax.fori_loop(0, DNF // DK, group, 0)
    _copy_idx(dst_all, DNF * DC, it, DTAIL)
    pltpu.sync_copy(ones_v.at[pl.ds(0, DTAIL)], acc_sh.at[it], add=True)
    plsc.subcore_barrier()
    pltpu.sync_copy(acc_sh.at[pl.ds(sid * RPT, RPT)],
                    out_hbm.at[pl.ds(cid * NPAD + sid * RPT, RPT)])


CF = 96
NFULL = EPT // CF
TAIL = EPT - NFULL * CF


@functools.partial(
    pl.kernel,
    mesh=_mesh,
    out_type=jax.ShapeDtypeStruct((NC * NPAD, D), jnp.float32),
    scratch_types=[
        pltpu.VMEM((EPT,), jnp.int32),
        pltpu.VMEM((EPT,), jnp.int32),
        pltpu.VMEM((CF,), jnp.int32),
        pltpu.VMEM((TAIL,), jnp.int32),
        pltpu.VMEM((CF, D), jnp.float32),
        pltpu.VMEM((CF, D), jnp.float32),
        pltpu.VMEM_SHARED((NPAD, D), jnp.float32),
        pltpu.SemaphoreType.DMA,
        pltpu.SemaphoreType.DMA,
    ],
)
def _sc_mp(g_hbm, src_hbm, dst_hbm, zeros_hbm, out_hbm,
           src_all, dst_all, sidx, idx_t, rows0, rows1, acc_sh,
           sem0, sem1):
    cid = lax.axis_index("c")
    sid = lax.axis_index("s")
    base = cid * EPC + sid * EPT
    pltpu.sync_copy(src_hbm.at[pl.ds(base, EPT)], src_all)
    pltpu.sync_copy(dst_hbm.at[pl.ds(base, EPT)], dst_all)
    pltpu.sync_copy(zeros_hbm.at[pl.ds(sid * RPT, RPT)],
                    acc_sh.at[pl.ds(sid * RPT, RPT)])
    plsc.subcore_barrier()

    rows = (rows0, rows1)
    sems = (sem0, sem1)

    def issue_gather(j, p):
        return pltpu.async_copy(g_hbm.at[src_all.at[pl.ds(j * CF, CF)]],
                                rows[p], sems[p])

    def scatter(j, p):
        _copy_idx(dst_all, j * CF, sidx, CF)
        pltpu.sync_copy(rows[p], acc_sh.at[sidx], add=True)

    issue_gather(0, 0).wait()

    def pair(k, carry):
        j = 2 * k + 1
        h = issue_gather(j, 1)
        scatter(j - 1, 0)
        h.wait()
        h = issue_gather(j + 1, 0)
        scatter(j, 1)
        h.wait()
        return carry

    lax.fori_loop(0, (NFULL - 2) // 2, pair, 0)
    h = issue_gather(NFULL - 1, 1)
    scatter(NFULL - 2, 0)
    h.wait()
    scatter(NFULL - 1, 1)
    pltpu.async_copy(g_hbm.at[src_all.at[pl.ds(NFULL * CF, TAIL)]],
                     rows0.at[pl.ds(0, TAIL)], sems[0]).wait()
    _copy_idx(dst_all, NFULL * CF, idx_t, TAIL)
    pltpu.sync_copy(rows0.at[pl.ds(0, TAIL)], acc_sh.at[idx_t], add=True)

    plsc.subcore_barrier()
    pltpu.sync_copy(acc_sh.at[pl.ds(sid * RPT, RPT)],
                    out_hbm.at[pl.ds(cid * NPAD + sid * RPT, RPT)])


def _mm0_body(x_ref, w_ref, degp_ref, g_ref):
    deg = 1.0 + degp_ref[0, :N, :1] + degp_ref[1, :N, :1]
    dinv = 1.0 / jnp.sqrt(deg)
    g_ref[...] = dinv * jnp.dot(x_ref[...], w_ref[...],
                                preferred_element_type=jnp.float32)


_mm0 = pl.pallas_call(
    _mm0_body,
    out_shape=jax.ShapeDtypeStruct((N, D), jnp.float32),
)


def _epi_body(with_next, s_ref, g_ref, degp_ref, b_ref, gamma_ref, beta_ref,
              batch_ref, w_ref, pool_ref, gnext_ref=None):
    deg = 1.0 + degp_ref[0, :N, :1] + degp_ref[1, :N, :1]
    dinv = 1.0 / jnp.sqrt(deg)
    s = s_ref[0, :N] + s_ref[1, :N] + g_ref[...]
    u = jnp.maximum(dinv * s + b_ref[...], 0.0)
    mean = jnp.mean(u, axis=0, keepdims=True)
    var = jnp.mean((u - mean) ** 2, axis=0, keepdims=True)
    h = gamma_ref[...] * (u - mean) / jnp.sqrt(var + EPS) + beta_ref[...]
    sel = (batch_ref[...] ==
           lax.broadcasted_iota(jnp.int32, (G, N), 0)).astype(jnp.float32)
    pool_ref[...] = jnp.dot(sel, h, preferred_element_type=jnp.float32,
                            precision=lax.Precision.HIGHEST)
    if with_next:
        gnext_ref[...] = dinv * jnp.dot(h, w_ref[...],
                                        preferred_element_type=jnp.float32)


_epi_mid = pl.pallas_call(
    functools.partial(_epi_body, True),
    out_shape=(jax.ShapeDtypeStruct((G, D), jnp.float32),
               jax.ShapeDtypeStruct((N, D), jnp.float32)),
)

_epi_last = pl.pallas_call(
    functools.partial(_epi_body, False),
    out_shape=jax.ShapeDtypeStruct((G, D), jnp.float32),
)


def kernel(x, edge_index, batch,
           W0, b0, gamma0, beta0,
           W1, b1, gamma1, beta1,
           W2, b2, gamma2, beta2):
    src = edge_index[0]
    dst = edge_index[1]
    ones_c = jnp.ones((DC, DEGW), jnp.float32)
    zeros_deg = jnp.zeros((NPAD, DEGW), jnp.float32)
    zeros_nd = jnp.zeros((NPAD, D), jnp.float32)
    batch2 = batch.reshape(1, N)

    degp = _sc_deg(dst, ones_c, zeros_deg).reshape(NC, NPAD, DEGW)

    g = _mm0(x, W0, degp)

    params = [(b0, gamma0, beta0, W1), (b1, gamma1, beta1, W2),
              (b2, gamma2, beta2, None)]
    pools = []
    for li, (b, gamma, beta, Wn) in enumerate(params):
        s = _sc_mp(g, src, dst, zeros_nd).reshape(NC, NPAD, D)
        b2d = b.reshape(1, D)
        g2d = gamma.reshape(1, D)
        bt2d = beta.reshape(1, D)
        if Wn is not None:
            pool, g = _epi_mid(s, g, degp, b2d, g2d, bt2d, batch2, Wn)
        else:
            wd = jnp.zeros((D, D), jnp.float32)
            pool = _epi_last(s, g, degp, b2d, g2d, bt2d, batch2, wd)
        pools.append(pool)

    return jnp.concatenate(pools, axis=1)
